# baseline (speedup 1.0000x reference)
import jax
import jax.numpy as jnp
from jax import lax
from jax.experimental import pallas as pl
from jax.experimental.pallas import tpu as pltpu
from jax.experimental.pallas import tpu_sc as plsc

NUM_SHELVES = 100000
D = 64
BATCH = 16384

NW = 32

CT = 782
CT_FULL = 781
CT_PER_W = 25
VROWS = 100008
PACKED_ROWS = VROWS // 2

ROWS_PER_W = BATCH // NW
NCHUNK = 4
CHUNK = ROWS_PER_W // NCHUNK
NGROUP = CHUNK // 16


def _splat(vec, lane):
    return lax.gather(
        vec, jnp.full((16, 1), lane, jnp.int32),
        dimension_numbers=lax.GatherDimensionNumbers(
            offset_dims=(), collapsed_slice_dims=(0,), start_index_map=(0,)),
        slice_sizes=(1,),
        mode=lax.GatherScatterMode.PROMISE_IN_BOUNDS)


def _pick(vec, lane_idx):
    return lax.gather(
        vec, lane_idx[:, None],
        dimension_numbers=lax.GatherDimensionNumbers(
            offset_dims=(), collapsed_slice_dims=(0,), start_index_map=(0,)),
        slice_sizes=(1,),
        mode=lax.GatherScatterMode.PROMISE_IN_BOUNDS)


def _tr_body(wt_hbm, wtail_hbm, pk_hbm, in_a, in_b, out_a, out_b,
             la, lb, sa, sb):
    wid = lax.axis_index("s") * 2 + lax.axis_index("c")
    iota = lax.iota(jnp.int32, 16)
    ibufs = (in_a, in_b)
    obufs = (out_a, out_b)
    lsems = (la, lb)
    ssems = (sa, sb)
    NR = 26

    def cidx(r):
        return jnp.minimum(wid * NR + r, CT_FULL - 1)

    def fire_load(r, p):
        pltpu.async_copy(
            wt_hbm.at[:, pl.ds(cidx(r) * 128, 128)], ibufs[p], lsems[p])

    def wait_load(p):
        pltpu.make_async_copy(
            wt_hbm.at[:, pl.ds(0, 128)], ibufs[p], lsems[p]).wait()

    def fire_store(r, p):
        pltpu.async_copy(
            obufs[p], pk_hbm.at[pl.ds(cidx(r) * 64, 64)], ssems[p])

    def wait_store(p):
        pltpu.make_async_copy(
            obufs[p], pk_hbm.at[pl.ds(0, 64)], ssems[p]).wait()

    def transpose_block(src, dst):
        for pl_ in range(64):
            for h in range(2):
                il = 2 * pl_ + h
                colv = jnp.full((16,), il, jnp.int32)
                for dg in range(4):
                    v = plsc.load_gather(src, [16 * dg + iota, colv])
                    dst[pl_, pl.ds(h * D + 16 * dg, 16)] = v

    fire_load(0, 0)

    def two_rounds(it, _):
        r0 = 2 * it
        for p in range(2):
            r = r0 + p
            wait_load(p)
            pl.when(r + 1 < NR)(lambda: fire_load(r + 1, 1 - p))
            pl.when(r >= 2)(lambda: wait_store(p))
            transpose_block(ibufs[p], obufs[p])
            fire_store(r, p)
        return 0

    lax.fori_loop(0, NR // 2, two_rounds, 0)
    wait_store(0)
    wait_store(1)

    @pl.when(wid == 0)
    def _():
        pltpu.async_copy(wtail_hbm, pk_hbm.at[pl.ds(49984, 20)], sa).wait()


def _gt_body(w_hbm, iflat_hbm, out_hbm,
             iflat_v, i0_v, i1_v, i2_v, recips_v,
             ra0, ra1, ra2, rb0, rb1, rb2,
             ga0, ga1, ga2, gb0, gb1, gb2, oa, ob):
    wid = lax.axis_index("s") * 2 + lax.axis_index("c")
    base = wid * ROWS_PER_W

    pltpu.sync_copy(iflat_hbm.at[pl.ds(wid * (ROWS_PER_W * 3),
                                       ROWS_PER_W * 3)], iflat_v)

    iota = lax.iota(jnp.int32, 16)
    one = jnp.float32(1.0)
    zero = jnp.float32(0.0)

    def pre_body(t, _):
        fb = t * 48
        v0 = iflat_v[pl.ds(fb, 16)]
        v1 = iflat_v[pl.ds(fb + 16, 16)]
        v2 = iflat_v[pl.ds(fb + 32, 16)]
        c = t // NGROUP
        col = (t % NGROUP) * 16
        csl = pl.ds(col, 16)
        cols = []
        for j in range(3):
            flat = iota * 3 + j
            lane = lax.bitwise_and(flat, 15)
            src = lax.shift_right_logical(flat, 4)
            g0 = _pick(v0, lane)
            g1 = _pick(v1, lane)
            g2 = _pick(v2, lane)
            cols.append(jnp.where(src == 0, g0, jnp.where(src == 1, g1, g2)))
        i0_v[c, csl] = cols[0]
        i1_v[c, csl] = cols[1]
        i2_v[c, csl] = cols[2]
        cnt = (jnp.where(cols[0] != 0, one, zero)
               + jnp.where(cols[1] != 0, one, zero)
               + jnp.where(cols[2] != 0, one, zero))
        recips_v[c, csl] = one / jnp.maximum(cnt, one)
        return 0

    lax.fori_loop(0, NGROUP * NCHUNK, pre_body, 0)

    rbufs = ((ra0, ra1, ra2), (rb0, rb1, rb2))
    gsems = ((ga0, ga1, ga2), (gb0, gb1, gb2))
    osems = (oa, ob)

    def fire(j, s):
        r0, r1, r2 = rbufs[s]
        s0, s1, s2 = gsems[s]
        return (pltpu.async_copy(w_hbm.at[i0_v.at[j]], r0, s0),
                pltpu.async_copy(w_hbm.at[i1_v.at[j]], r1, s1),
                pltpu.async_copy(w_hbm.at[i2_v.at[j]], r2, s2))

    def compute(j, s):
        r0, r1, r2 = rbufs[s]

        def g_body(g, _):
            recip16 = recips_v[j, pl.ds(g * 16, 16)]
            for b in range(16):
                rb = _splat(recip16, b)
                row = g * 16 + b
                for dg in range(4):
                    dsl = pl.ds(dg * 16, 16)
                    r0[row, dsl] = (r0[row, dsl] + r1[row, dsl]
                                    + r2[row, dsl]) * rb
            return 0

        lax.fori_loop(0, NGROUP, g_body, 0)

    out_pending = [None, None]
    gat_pending = [None, None]
    gat_pending[0] = fire(0, 0)
    for j in range(NCHUNK):
        s = j & 1
        if j + 1 < NCHUNK:
            s2 = 1 - s
            if out_pending[s2] is not None:
                out_pending[s2].wait()
                out_pending[s2] = None
            gat_pending[s2] = fire(j + 1, s2)
        for h in gat_pending[s]:
            h.wait()
        compute(j, s)
        out_pending[s] = pltpu.async_copy(
            rbufs[s][0],
            out_hbm.at[pl.ds(base + j * CHUNK, CHUNK)],
            osems[s])
    for s in range(2):
        if out_pending[s] is not None:
            out_pending[s].wait()


@jax.jit
def _shelf_embed(wt, wtail, iflat):
    mesh = plsc.VectorSubcoreMesh(core_axis_name="c", subcore_axis_name="s")
    tr = pl.kernel(
        _tr_body,
        out_type=jax.ShapeDtypeStruct((PACKED_ROWS, 128), jnp.float32),
        mesh=mesh,
        scratch_types=[
            pltpu.VMEM((D, 128), jnp.float32),
            pltpu.VMEM((D, 128), jnp.float32),
            pltpu.VMEM((64, 128), jnp.float32),
            pltpu.VMEM((64, 128), jnp.float32),
            pltpu.SemaphoreType.DMA,
            pltpu.SemaphoreType.DMA,
            pltpu.SemaphoreType.DMA,
            pltpu.SemaphoreType.DMA,
        ],
        compiler_params=pltpu.CompilerParams(use_tc_tiling_on_sc=True,
                                             needs_layout_passes=False),
    )
    packed = tr(wt, wtail)
    w64 = packed.reshape(VROWS, D)

    gt = pl.kernel(
        _gt_body,
        out_type=jax.ShapeDtypeStruct((BATCH, D), jnp.float32),
        mesh=mesh,
        scratch_types=[
            pltpu.VMEM((ROWS_PER_W * 3,), jnp.int32),
            pltpu.VMEM((NCHUNK, CHUNK), jnp.int32),
            pltpu.VMEM((NCHUNK, CHUNK), jnp.int32),
            pltpu.VMEM((NCHUNK, CHUNK), jnp.int32),
            pltpu.VMEM((NCHUNK, CHUNK), jnp.float32),
            pltpu.VMEM((CHUNK, D), jnp.float32),
            pltpu.VMEM((CHUNK, D), jnp.float32),
            pltpu.VMEM((CHUNK, D), jnp.float32),
            pltpu.VMEM((CHUNK, D), jnp.float32),
            pltpu.VMEM((CHUNK, D), jnp.float32),
            pltpu.VMEM((CHUNK, D), jnp.float32),
            pltpu.SemaphoreType.DMA,
            pltpu.SemaphoreType.DMA,
            pltpu.SemaphoreType.DMA,
            pltpu.SemaphoreType.DMA,
            pltpu.SemaphoreType.DMA,
            pltpu.SemaphoreType.DMA,
            pltpu.SemaphoreType.DMA,
            pltpu.SemaphoreType.DMA,
        ],
        compiler_params=pltpu.CompilerParams(use_tc_tiling_on_sc=False),
    )
    return gt(w64, iflat)


def kernel(shelf_indices, weight):
    wt = weight.T
    wtail = jnp.pad(weight[99968:], ((0, 7), (0, 0))).reshape(20, 128)
    iflat = shelf_indices.astype(jnp.int32).reshape(BATCH * 3)
    return _shelf_embed(wt, wtail, iflat)


# speedup vs baseline: 2.2605x; 2.2605x over previous
import jax
import jax.numpy as jnp
from jax import lax
from jax.experimental import pallas as pl
from jax.experimental.pallas import tpu as pltpu
from jax.experimental.pallas import tpu_sc as plsc

NUM_SHELVES = 100000
D = 64
BATCH = 16384

NW = 32
ROWS_PER_W = BATCH // NW
NCHUNK = 8
CHUNK = ROWS_PER_W // NCHUNK
NGROUP = CHUNK // 16
IPC = CHUNK * 3


def _splat(vec, lane):
    return lax.gather(
        vec, jnp.full((16, 1), lane, jnp.int32),
        dimension_numbers=lax.GatherDimensionNumbers(
            offset_dims=(), collapsed_slice_dims=(0,), start_index_map=(0,)),
        slice_sizes=(1,),
        mode=lax.GatherScatterMode.PROMISE_IN_BOUNDS)


def _pick(vec, lane_idx):
    return lax.gather(
        vec, lane_idx[:, None],
        dimension_numbers=lax.GatherDimensionNumbers(
            offset_dims=(), collapsed_slice_dims=(0,), start_index_map=(0,)),
        slice_sizes=(1,),
        mode=lax.GatherScatterMode.PROMISE_IN_BOUNDS)


def _sc_body(w_hbm, iflat_hbm, out_hbm,
             iflat_v, recips_v, rows_a, rows_b, out_a, out_b,
             ga, gb, oa, ob):
    wid = lax.axis_index("s") * 2 + lax.axis_index("c")
    base = wid * ROWS_PER_W

    pltpu.sync_copy(iflat_hbm.at[pl.ds(wid * (ROWS_PER_W * 3),
                                       ROWS_PER_W * 3)], iflat_v)

    iota = lax.iota(jnp.int32, 16)
    one = jnp.float32(1.0)
    zero = jnp.float32(0.0)

    def pre_body(t, _):
        fb = t * 48
        v0 = iflat_v[pl.ds(fb, 16)]
        v1 = iflat_v[pl.ds(fb + 16, 16)]
        v2 = iflat_v[pl.ds(fb + 32, 16)]
        c = t // NGROUP
        col = (t % NGROUP) * 16
        cols = []
        for j in range(3):
            flat = iota * 3 + j
            lane = lax.bitwise_and(flat, 15)
            src = lax.shift_right_logical(flat, 4)
            g0 = _pick(v0, lane)
            g1 = _pick(v1, lane)
            g2 = _pick(v2, lane)
            cols.append(jnp.where(src == 0, g0, jnp.where(src == 1, g1, g2)))
        cnt = (jnp.where(cols[0] != 0, one, zero)
               + jnp.where(cols[1] != 0, one, zero)
               + jnp.where(cols[2] != 0, one, zero))
        recips_v[c, pl.ds(col, 16)] = one / jnp.maximum(cnt, one)
        return 0

    lax.fori_loop(0, NGROUP * NCHUNK, pre_body, 0)

    rbufs = (rows_a, rows_b)
    gsems = (ga, gb)
    obufs = (out_a, out_b)
    osems = (oa, ob)

    def fire(j, s):
        sem = gsems[s]
        rows = rbufs[s]

        def issue(g, _):
            v = iflat_v[pl.ds(j * IPC + g * 16, 16)]
            for l in range(16):
                pltpu.async_copy(w_hbm.at[pl.ds(v[l], 1)],
                                 rows.at[pl.ds(g * 16 + l, 1)], sem)
            return 0

        lax.fori_loop(0, IPC // 16, issue, 0)

    def drain(s):
        def dwait(r, _):
            pltpu.make_async_copy(w_hbm.at[pl.ds(0, 1)],
                                  rbufs[s].at[pl.ds(0, 1)], gsems[s]).wait()
            return 0

        lax.fori_loop(0, IPC, dwait, 0)

    def compute(j, s):
        rows = rbufs[s]
        outb = obufs[s]

        def g_body(g, _):
            recip16 = recips_v[j, pl.ds(g * 16, 16)]
            for b in range(16):
                rb = _splat(recip16, b)
                row = g * 16 + b
                fb = 3 * row
                for dg in range(4):
                    dsl = pl.ds(dg * 16, 16)
                    outb[row, dsl] = (rows[fb, dsl] + rows[fb + 1, dsl]
                                      + rows[fb + 2, dsl]) * rb
            return 0

        lax.fori_loop(0, NGROUP, g_body, 0)

    out_pending = [None, None]
    fire(0, 0)
    for j in range(NCHUNK):
        s = j & 1
        drain(s)
        if j + 1 < NCHUNK:
            s2 = 1 - s
            if out_pending[s2] is not None:
                out_pending[s2].wait()
                out_pending[s2] = None
            fire(j + 1, s2)
        compute(j, s)
        out_pending[s] = pltpu.async_copy(
            obufs[s], out_hbm.at[pl.ds(base + j * CHUNK, CHUNK)], osems[s])
    for s in range(2):
        if out_pending[s] is not None:
            out_pending[s].wait()


@jax.jit
def _shelf_embed(weight, iflat):
    mesh = plsc.VectorSubcoreMesh(core_axis_name="c", subcore_axis_name="s")
    fn = pl.kernel(
        _sc_body,
        out_type=jax.ShapeDtypeStruct((BATCH, D), jnp.float32),
        mesh=mesh,
        scratch_types=[
            pltpu.VMEM((ROWS_PER_W * 3,), jnp.int32),
            pltpu.VMEM((NCHUNK, CHUNK), jnp.float32),
            pltpu.VMEM((IPC, D), jnp.float32),
            pltpu.VMEM((IPC, D), jnp.float32),
            pltpu.VMEM((CHUNK, D), jnp.float32),
            pltpu.VMEM((CHUNK, D), jnp.float32),
            pltpu.SemaphoreType.DMA,
            pltpu.SemaphoreType.DMA,
            pltpu.SemaphoreType.DMA,
            pltpu.SemaphoreType.DMA,
        ],
        compiler_params=pltpu.CompilerParams(use_tc_tiling_on_sc=True,
                                             needs_layout_passes=False),
    )
    return fn(weight, iflat)


def kernel(shelf_indices, weight):
    iflat = shelf_indices.astype(jnp.int32).reshape(BATCH * 3)
    return _shelf_embed(weight, iflat)


# speedup vs baseline: 2.2920x; 1.0139x over previous
import jax
import jax.numpy as jnp
from jax import lax
from jax.experimental import pallas as pl
from jax.experimental.pallas import tpu as pltpu
from jax.experimental.pallas import tpu_sc as plsc

NUM_SHELVES = 100000
D = 64
BATCH = 16384

NW = 32
ROWS_PER_W = BATCH // NW
NCHUNK = 8
CHUNK = ROWS_PER_W // NCHUNK
NGROUP = CHUNK // 16


def _splat(vec, lane):
    return lax.gather(
        vec, jnp.full((16, 1), lane, jnp.int32),
        dimension_numbers=lax.GatherDimensionNumbers(
            offset_dims=(), collapsed_slice_dims=(0,), start_index_map=(0,)),
        slice_sizes=(1,),
        mode=lax.GatherScatterMode.PROMISE_IN_BOUNDS)


def _pick(vec, lane_idx):
    return lax.gather(
        vec, lane_idx[:, None],
        dimension_numbers=lax.GatherDimensionNumbers(
            offset_dims=(), collapsed_slice_dims=(0,), start_index_map=(0,)),
        slice_sizes=(1,),
        mode=lax.GatherScatterMode.PROMISE_IN_BOUNDS)


def _sc_body(w_hbm, iflat_hbm, out_hbm,
             iflat_v, i0_v, i1_v, i2_v, recips_v,
             ra0, ra1, ra2, rb0, rb1, rb2, out_a, out_b,
             ga0, ga1, ga2, gb0, gb1, gb2, oa, ob):
    wid = lax.axis_index("s") * 2 + lax.axis_index("c")
    base = wid * ROWS_PER_W

    pltpu.sync_copy(iflat_hbm.at[pl.ds(wid * (ROWS_PER_W * 3),
                                       ROWS_PER_W * 3)], iflat_v)

    iota = lax.iota(jnp.int32, 16)
    one = jnp.float32(1.0)
    zero = jnp.float32(0.0)

    def pre_body(t, _):
        fb = t * 48
        v0 = iflat_v[pl.ds(fb, 16)]
        v1 = iflat_v[pl.ds(fb + 16, 16)]
        v2 = iflat_v[pl.ds(fb + 32, 16)]
        c = t // 4
        col = (t % 4) * 16
        csl = pl.ds(col, 16)
        cols = []
        for j in range(3):
            flat = iota * 3 + j
            lane = lax.bitwise_and(flat, 15)
            src = lax.shift_right_logical(flat, 4)
            g0 = _pick(v0, lane)
            g1 = _pick(v1, lane)
            g2 = _pick(v2, lane)
            cols.append(jnp.where(src == 0, g0, jnp.where(src == 1, g1, g2)))
        i0_v[c, csl] = cols[0]
        i1_v[c, csl] = cols[1]
        i2_v[c, csl] = cols[2]
        cnt = (jnp.where(cols[0] != 0, one, zero)
               + jnp.where(cols[1] != 0, one, zero)
               + jnp.where(cols[2] != 0, one, zero))
        recips_v[c, csl] = one / jnp.maximum(cnt, one)
        return 0

    lax.fori_loop(0, (NGROUP) * NCHUNK, pre_body, 0)

    rbufs = ((ra0, ra1, ra2), (rb0, rb1, rb2))
    gsems = ((ga0, ga1, ga2), (gb0, gb1, gb2))
    obufs = (out_a, out_b)
    osems = (oa, ob)

    def fire(j, s):
        r0, r1, r2 = rbufs[s]
        s0, s1, s2 = gsems[s]
        return (pltpu.async_copy(w_hbm.at[i0_v.at[j]], r0, s0),
                pltpu.async_copy(w_hbm.at[i1_v.at[j]], r1, s1),
                pltpu.async_copy(w_hbm.at[i2_v.at[j]], r2, s2))

    def compute(j, s):
        r0, r1, r2 = rbufs[s]
        outb = obufs[s]

        def g_body(g, _):
            recip16 = recips_v[j, pl.ds(g * 16, 16)]
            for b in range(16):
                rb = _splat(recip16, b)
                row = g * 16 + b
                for dg in range(4):
                    dsl = pl.ds(dg * 16, 16)
                    outb[row, dsl] = (r0[row, dsl] + r1[row, dsl]
                                      + r2[row, dsl]) * rb
            return 0

        lax.fori_loop(0, NGROUP, g_body, 0)

    out_pending = [None, None]
    gat_pending = [None, None]
    gat_pending[0] = fire(0, 0)
    for j in range(NCHUNK):
        s = j & 1
        if j + 1 < NCHUNK:
            s2 = 1 - s
            if out_pending[s2] is not None:
                out_pending[s2].wait()
                out_pending[s2] = None
            gat_pending[s2] = fire(j + 1, s2)
        for h in gat_pending[s]:
            h.wait()
        compute(j, s)
        out_pending[s] = pltpu.async_copy(
            obufs[s],
            out_hbm.at[pl.ds(base + j * CHUNK, CHUNK)],
            osems[s])
    for s in range(2):
        if out_pending[s] is not None:
            out_pending[s].wait()


@jax.jit
def _shelf_embed(wpad, iflat):
    mesh = plsc.VectorSubcoreMesh(core_axis_name="c", subcore_axis_name="s")
    fn = pl.kernel(
        _sc_body,
        out_type=jax.ShapeDtypeStruct((BATCH, D), jnp.float32),
        mesh=mesh,
        scratch_types=[
            pltpu.VMEM((ROWS_PER_W * 3,), jnp.int32),
            pltpu.VMEM((NCHUNK, CHUNK), jnp.int32),
            pltpu.VMEM((NCHUNK, CHUNK), jnp.int32),
            pltpu.VMEM((NCHUNK, CHUNK), jnp.int32),
            pltpu.VMEM((NCHUNK, CHUNK), jnp.float32),
            pltpu.VMEM((CHUNK, 128), jnp.float32),
            pltpu.VMEM((CHUNK, 128), jnp.float32),
            pltpu.VMEM((CHUNK, 128), jnp.float32),
            pltpu.VMEM((CHUNK, 128), jnp.float32),
            pltpu.VMEM((CHUNK, 128), jnp.float32),
            pltpu.VMEM((CHUNK, 128), jnp.float32),
            pltpu.VMEM((CHUNK, D), jnp.float32),
            pltpu.VMEM((CHUNK, D), jnp.float32),
            pltpu.SemaphoreType.DMA,
            pltpu.SemaphoreType.DMA,
            pltpu.SemaphoreType.DMA,
            pltpu.SemaphoreType.DMA,
            pltpu.SemaphoreType.DMA,
            pltpu.SemaphoreType.DMA,
            pltpu.SemaphoreType.DMA,
            pltpu.SemaphoreType.DMA,
        ],
        compiler_params=pltpu.CompilerParams(use_tc_tiling_on_sc=True),
    )
    return fn(wpad, iflat)


def kernel(shelf_indices, weight):
    wpad = jnp.pad(weight, ((0, 0), (0, 128 - D)))
    iflat = shelf_indices.astype(jnp.int32).reshape(BATCH * 3)
    return _shelf_embed(wpad, iflat)


# speedup vs baseline: 2.3035x; 1.0050x over previous
import jax
import jax.numpy as jnp
from jax import lax
from jax.experimental import pallas as pl
from jax.experimental.pallas import tpu as pltpu
from jax.experimental.pallas import tpu_sc as plsc

NUM_SHELVES = 100000
D = 64
BATCH = 16384

NW = 32
ROWS_PER_W = BATCH // NW
NCHUNK = 8
CHUNK = ROWS_PER_W // NCHUNK
NGROUP = CHUNK // 16


def _splat(vec, lane):
    return lax.gather(
        vec, jnp.full((16, 1), lane, jnp.int32),
        dimension_numbers=lax.GatherDimensionNumbers(
            offset_dims=(), collapsed_slice_dims=(0,), start_index_map=(0,)),
        slice_sizes=(1,),
        mode=lax.GatherScatterMode.PROMISE_IN_BOUNDS)


def _pick(vec, lane_idx):
    return lax.gather(
        vec, lane_idx[:, None],
        dimension_numbers=lax.GatherDimensionNumbers(
            offset_dims=(), collapsed_slice_dims=(0,), start_index_map=(0,)),
        slice_sizes=(1,),
        mode=lax.GatherScatterMode.PROMISE_IN_BOUNDS)


def _sc_body(w_hbm, iflat_hbm, out_hbm,
             iflat_v, i0_v, i1_v, i2_v, recips_v,
             ra0, ra1, ra2, rb0, rb1, rb2,
             ga0, ga1, ga2, gb0, gb1, gb2, oa, ob):
    wid = lax.axis_index("s") * 2 + lax.axis_index("c")
    base = wid * ROWS_PER_W

    pltpu.sync_copy(iflat_hbm.at[pl.ds(wid * (ROWS_PER_W * 3),
                                       ROWS_PER_W * 3)], iflat_v)

    iota = lax.iota(jnp.int32, 16)
    one = jnp.float32(1.0)
    zero = jnp.float32(0.0)

    def pre_body(t, _):
        fb = t * 48
        v0 = iflat_v[pl.ds(fb, 16)]
        v1 = iflat_v[pl.ds(fb + 16, 16)]
        v2 = iflat_v[pl.ds(fb + 32, 16)]
        c = t // 4
        col = (t % 4) * 16
        csl = pl.ds(col, 16)
        cols = []
        for j in range(3):
            flat = iota * 3 + j
            lane = lax.bitwise_and(flat, 15)
            src = lax.shift_right_logical(flat, 4)
            g0 = _pick(v0, lane)
            g1 = _pick(v1, lane)
            g2 = _pick(v2, lane)
            cols.append(jnp.where(src == 0, g0, jnp.where(src == 1, g1, g2)))
        i0_v[c, csl] = cols[0]
        i1_v[c, csl] = cols[1]
        i2_v[c, csl] = cols[2]
        cnt = (jnp.where(cols[0] != 0, one, zero)
               + jnp.where(cols[1] != 0, one, zero)
               + jnp.where(cols[2] != 0, one, zero))
        recips_v[c, csl] = one / jnp.maximum(cnt, one)
        return 0

    lax.fori_loop(0, (NGROUP) * NCHUNK, pre_body, 0)

    rbufs = ((ra0, ra1, ra2), (rb0, rb1, rb2))
    gsems = ((ga0, ga1, ga2), (gb0, gb1, gb2))
    osems = (oa, ob)

    def fire(j, s):
        r0, r1, r2 = rbufs[s]
        s0, s1, s2 = gsems[s]
        return (pltpu.async_copy(w_hbm.at[i0_v.at[j]], r0, s0),
                pltpu.async_copy(w_hbm.at[i1_v.at[j]], r1, s1),
                pltpu.async_copy(w_hbm.at[i2_v.at[j]], r2, s2))

    def compute(j, s):
        r0, r1, r2 = rbufs[s]

        def g_body(g, _):
            recip16 = recips_v[j, pl.ds(g * 16, 16)]
            for b in range(16):
                rb = _splat(recip16, b)
                row = g * 16 + b
                for dg in range(4):
                    dsl = pl.ds(dg * 16, 16)
                    r0[row, dsl] = (r0[row, dsl] + r1[row, dsl]
                                    + r2[row, dsl]) * rb
            return 0

        lax.fori_loop(0, NGROUP, g_body, 0)

    out_pending = [None, None]
    gat_pending = [None, None]
    gat_pending[0] = fire(0, 0)
    for j in range(NCHUNK):
        s = j & 1
        if j + 1 < NCHUNK:
            s2 = 1 - s
            if out_pending[s2] is not None:
                out_pending[s2].wait()
                out_pending[s2] = None
            gat_pending[s2] = fire(j + 1, s2)
        for h in gat_pending[s]:
            h.wait()
        compute(j, s)
        out_pending[s] = pltpu.async_copy(
            rbufs[s][0],
            out_hbm.at[pl.ds(base + j * CHUNK, CHUNK)],
            osems[s])
    for s in range(2):
        if out_pending[s] is not None:
            out_pending[s].wait()


@jax.jit
def _shelf_embed(wpad, iflat):
    mesh = plsc.VectorSubcoreMesh(core_axis_name="c", subcore_axis_name="s")
    fn = pl.kernel(
        _sc_body,
        out_type=jax.ShapeDtypeStruct((BATCH, 128), jnp.float32),
        mesh=mesh,
        scratch_types=[
            pltpu.VMEM((ROWS_PER_W * 3,), jnp.int32),
            pltpu.VMEM((NCHUNK, CHUNK), jnp.int32),
            pltpu.VMEM((NCHUNK, CHUNK), jnp.int32),
            pltpu.VMEM((NCHUNK, CHUNK), jnp.int32),
            pltpu.VMEM((NCHUNK, CHUNK), jnp.float32),
            pltpu.VMEM((CHUNK, 128), jnp.float32),
            pltpu.VMEM((CHUNK, 128), jnp.float32),
            pltpu.VMEM((CHUNK, 128), jnp.float32),
            pltpu.VMEM((CHUNK, 128), jnp.float32),
            pltpu.VMEM((CHUNK, 128), jnp.float32),
            pltpu.VMEM((CHUNK, 128), jnp.float32),
            pltpu.SemaphoreType.DMA,
            pltpu.SemaphoreType.DMA,
            pltpu.SemaphoreType.DMA,
            pltpu.SemaphoreType.DMA,
            pltpu.SemaphoreType.DMA,
            pltpu.SemaphoreType.DMA,
            pltpu.SemaphoreType.DMA,
            pltpu.SemaphoreType.DMA,
        ],
        compiler_params=pltpu.CompilerParams(use_tc_tiling_on_sc=True),
    )
    return fn(wpad, iflat)


def kernel(shelf_indices, weight):
    wpad = jnp.pad(weight, ((0, 0), (0, 128 - D)))
    iflat = shelf_indices.astype(jnp.int32).reshape(BATCH * 3)
    return _shelf_embed(wpad, iflat)[:, :D]
